# Initial kernel scaffold; baseline (speedup 1.0000x reference)
import jax
import jax.numpy as jnp
from jax.experimental import pallas as pl


def kernel(node_features, indices, data, W, b):
    raise NotImplementedError("write your pallas kernel here")


# speedup vs baseline: 4.9032x; 4.9032x over previous
import functools

import jax
import jax.numpy as jnp
from jax import lax
from jax.experimental import pallas as pl
from jax.experimental.pallas import tpu as pltpu
from jax.experimental.pallas import tpu_sc as plsc

NC = 2
NS = 16
L = 16

BLK = 128


def _seg_pad(nseg):
    chunk = ((nseg + NS * 8 - 1) // (NS * 8)) * 8
    return chunk, chunk * NS


def _sc_pass(feat, gidx, sidx, data, nseg, with_scalar_sums):
    nrows, d = feat.shape
    nnz = data.shape[0]
    assert d % L == 0 and nnz % (BLK * NC) == 0
    nblk = nnz // BLK
    blk_per_core = nblk // NC
    chunk, seg_pad = _seg_pad(nseg)
    zlen = ((chunk + L - 1) // L) * L

    out_type = [jax.ShapeDtypeStruct((NC, seg_pad, d), jnp.float32)]
    scratch = [
        pltpu.VMEM((BLK,), jnp.int32),
        pltpu.VMEM((BLK,), jnp.int32),
        pltpu.VMEM((BLK,), jnp.float32),
        pltpu.VMEM((BLK, d), jnp.float32),
        pltpu.VMEM((zlen,), jnp.float32),
        pltpu.VMEM_SHARED((seg_pad, d), jnp.float32),
        pltpu.SemaphoreType.DMA,
    ]
    if with_scalar_sums:
        out_type += [jax.ShapeDtypeStruct((NC * seg_pad,), jnp.float32)] * 2
        scratch += [pltpu.VMEM_SHARED((seg_pad,), jnp.float32)] * 2

    mesh = plsc.VectorSubcoreMesh(core_axis_name="c", subcore_axis_name="s",
                                  num_cores=NC, num_subcores=NS)

    def body(*refs):
        if with_scalar_sums:
            (feat_h, gidx_h, sidx_h, data_h, part_h, dg_h, ds_h,
             idxg, idxs, dat, rows, zbuf, acc, sem, dg_sh, ds_sh) = refs
        else:
            (feat_h, gidx_h, sidx_h, data_h, part_h,
             idxg, idxs, dat, rows, zbuf, acc, sem) = refs

        c = lax.axis_index("c")
        s = lax.axis_index("s")

        zero = jnp.zeros((L,), jnp.float32)

        def zrow(i, _):
            for k in range(d // L):
                rows[i, pl.ds(k * L, L)] = zero
            return 0

        lax.fori_loop(0, BLK, zrow, 0)
        for k in range(zlen // L):
            zbuf[pl.ds(k * L, L)] = zero

        nchunks = chunk // BLK
        rem = chunk - nchunks * BLK
        for k in range(nchunks):
            pltpu.sync_copy(rows, acc.at[pl.ds(s * chunk + k * BLK, BLK)])
        if rem:
            pltpu.sync_copy(rows.at[pl.ds(0, rem)],
                            acc.at[pl.ds(s * chunk + nchunks * BLK, rem)])
        if with_scalar_sums:
            pltpu.sync_copy(zbuf.at[pl.ds(0, chunk)],
                            dg_sh.at[pl.ds(s * chunk, chunk)])
            pltpu.sync_copy(zbuf.at[pl.ds(0, chunk)],
                            ds_sh.at[pl.ds(s * chunk, chunk)])
        plsc.subcore_barrier()

        nb = blk_per_core // NS + jnp.where(s < blk_per_core % NS, 1, 0)

        def block_body(j, _):
            b = c * blk_per_core + j * NS + s
            base = b * BLK
            pltpu.sync_copy(gidx_h.at[pl.ds(base, BLK)], idxg)
            pltpu.sync_copy(sidx_h.at[pl.ds(base, BLK)], idxs)
            pltpu.sync_copy(data_h.at[pl.ds(base, BLK)], dat)
            pltpu.async_copy(feat_h.at[idxg], rows, sem).wait()

            def scale(g, _):
                chunk = dat[pl.ds(g * L, L)]
                for j in range(L):
                    bc = jnp.broadcast_to(chunk[j], (L,))
                    i = g * L + j
                    for k in range(d // L):
                        sl = pl.ds(k * L, L)
                        rows[i, sl] = rows[i, sl] * bc
                return 0

            lax.fori_loop(0, BLK // L, scale, 0)
            pltpu.sync_copy(rows, acc.at[idxs], add=True)
            if with_scalar_sums:
                pltpu.sync_copy(dat, dg_sh.at[idxg], add=True)
                pltpu.sync_copy(dat, ds_sh.at[idxs], add=True)
            return 0

        lax.fori_loop(0, nb, block_body, 0)
        plsc.subcore_barrier()

        pltpu.sync_copy(acc.at[pl.ds(s * chunk, chunk)],
                        part_h.at[c, pl.ds(s * chunk, chunk)])
        if with_scalar_sums:
            pltpu.sync_copy(dg_sh.at[pl.ds(s * chunk, chunk)],
                            zbuf.at[pl.ds(0, chunk)])
            pltpu.sync_copy(zbuf.at[pl.ds(0, chunk)],
                            dg_h.at[pl.ds(c * seg_pad + s * chunk, chunk)])
            pltpu.sync_copy(ds_sh.at[pl.ds(s * chunk, chunk)],
                            zbuf.at[pl.ds(0, chunk)])
            pltpu.sync_copy(zbuf.at[pl.ds(0, chunk)],
                            ds_h.at[pl.ds(c * seg_pad + s * chunk, chunk)])

    run = pl.kernel(body, out_type=out_type, mesh=mesh, scratch_types=scratch)
    return run(feat, gidx, sidx, data)


def _combine1(p, de, dv, W, b, nseg):
    def body(p_ref, de_ref, dv_ref, w_ref, b_ref, e_ref, dvi_ref):
        ssum = p_ref[0] + p_ref[1]
        des = de_ref[0] + de_ref[1]
        dvs = dv_ref[0] + dv_ref[1]
        e_pre = lax.dot_general(ssum, w_ref[...], (((1,), (1,)), ((), ())),
                                preferred_element_type=jnp.float32)
        e_pre = e_pre + des * b_ref[...]
        de_inv = jnp.where(des > 0, 1.0 / des, 0.0)
        e_ref[...] = e_pre * de_inv
        dvi_ref[...] = jnp.where(dvs > 0, 1.0 / dvs, 0.0)

    d = p.shape[2]
    grid_spec = pl.GridSpec(
        grid=(1,),
        in_specs=[
            pl.BlockSpec((NC, nseg, d), lambda i: (0, 0, 0)),
            pl.BlockSpec((NC, nseg, 1), lambda i: (0, 0, 0)),
            pl.BlockSpec((NC, nseg, 1), lambda i: (0, 0, 0)),
            pl.BlockSpec(W.shape, lambda i: (0, 0)),
            pl.BlockSpec((1, d), lambda i: (0, 0)),
        ],
        out_specs=(
            pl.BlockSpec((nseg, d), lambda i: (0, 0)),
            pl.BlockSpec((nseg, 1), lambda i: (0, 0)),
        ),
    )
    return pl.pallas_call(
        body,
        grid_spec=grid_spec,
        out_shape=(jax.ShapeDtypeStruct((nseg, d), jnp.float32),
                   jax.ShapeDtypeStruct((nseg, 1), jnp.float32)),
    )(p, de, dv, W, b.reshape(1, -1))


def _combine2(q, dvi, nseg):
    def body(q_ref, dvi_ref, o_ref):
        o_ref[...] = (q_ref[0] + q_ref[1]) * dvi_ref[...]

    d = q.shape[2]
    grid_spec = pl.GridSpec(
        grid=(1,),
        in_specs=[
            pl.BlockSpec((NC, nseg, d), lambda i: (0, 0, 0)),
            pl.BlockSpec((nseg, 1), lambda i: (0, 0)),
        ],
        out_specs=pl.BlockSpec((nseg, d), lambda i: (0, 0)),
    )
    return pl.pallas_call(
        body,
        grid_spec=grid_spec,
        out_shape=jax.ShapeDtypeStruct((nseg, d), jnp.float32),
    )(q, dvi)


def kernel(node_features, indices, data, W, b):
    n = node_features.shape[0]
    m = 10000
    v_idx = indices[:, 0]
    e_idx = indices[:, 1]

    p, dv_flat, de_flat = _sc_pass(node_features, v_idx, e_idx, data, m, True)
    _, seg_pad = _seg_pad(m)
    de = de_flat.reshape(NC, seg_pad, 1)
    dv = dv_flat.reshape(NC, seg_pad, 1)
    e_norm, dv_inv = _combine1(p, de, dv, W, b, m)
    q, = _sc_pass(e_norm, e_idx, v_idx, data, n, False)
    out = _combine2(q, dv_inv, n)
    return out
